# baseline (speedup 1.0000x reference)
import functools

import jax
import jax.numpy as jnp
from jax import lax
from jax.experimental import pallas as pl
from jax.experimental.pallas import tpu as pltpu
from jax.experimental.pallas import tpu_sc as plsc

B, L, D = 4, 2048, 1024
NC, NS, LANES = 2, 16, 16
NW = NC * NS
WPB = NW // B
RPW = L // WPB
CHUNK = 32
NCHUNK = RPW // CHUNK
G = D // LANES
JG = 8
NJG = G // JG


def _sc_body(hid, msk, pm, pa, pc,
             mi_v, mf_v, cnt_v, accm_v, acca_v, xb0, xb1, sem0, sem1):
    cid = lax.axis_index("c")
    sid = lax.axis_index("s")
    wid = sid * NC + cid
    b = wid // WPB
    j = wid % WPB
    r0 = j * RPW

    pltpu.sync_copy(msk.at[b, pl.ds(r0, RPW)], mi_v)

    def mconv(i, cnt):
        mi = mi_v[pl.ds(i * LANES, LANES)]
        mf = jnp.where(mi == 2, 1.0, 0.0).astype(jnp.float32)
        mf_v[pl.ds(i * LANES, LANES)] = mf
        return cnt + mf

    cnt = lax.fori_loop(0, RPW // LANES, mconv,
                        jnp.zeros((LANES,), jnp.float32))
    cnt_v[...] = cnt
    pltpu.sync_copy(cnt_v, pc.at[b, pl.ds(j * LANES, LANES)])

    def zbody(i, _):
        z = jnp.zeros((LANES,), jnp.float32)
        accm_v[pl.ds(i * LANES, LANES)] = z
        acca_v[pl.ds(i * LANES, LANES)] = z
        return 0

    lax.fori_loop(0, G, zbody, 0)

    bufs = (xb0, xb1)
    sems = (sem0, sem1)

    def dma(c, k):
        return pltpu.make_async_copy(
            hid.at[b, pl.ds(r0 + c * CHUNK, CHUNK)], bufs[k], sems[k])

    dma(0, 0).start()
    dma(1, 1).start()

    def chunk_pair(i, _):
        cc = i * 2
        for k in range(2):
            c = cc + k
            dma(c, k).wait()
            xb = bufs[k]

            def jg_body(g, _):
                base = g * (JG * LANES)
                accm0 = tuple(accm_v[pl.ds(base + t * LANES, LANES)]
                              for t in range(JG))
                acca0 = tuple(acca_v[pl.ds(base + t * LANES, LANES)]
                              for t in range(JG))

                def rg_body(rg, carry):
                    am, aa = carry
                    am = list(am)
                    aa = list(aa)
                    mvec = mf_v[pl.ds(c * CHUNK + rg * LANES, LANES)]
                    for li in range(LANES):
                        m = jnp.full((LANES,), mvec[li])
                        row = rg * LANES + li
                        for t in range(JG):
                            x = xb[row, pl.ds(base + t * LANES, LANES)]
                            am[t] = am[t] + m * x
                            aa[t] = aa[t] + x
                    return (tuple(am), tuple(aa))

                am, aa = lax.fori_loop(0, CHUNK // LANES, rg_body,
                                       (accm0, acca0))
                for t in range(JG):
                    accm_v[pl.ds(base + t * LANES, LANES)] = am[t]
                    acca_v[pl.ds(base + t * LANES, LANES)] = aa[t]
                return 0

            lax.fori_loop(0, NJG, jg_body, 0)

            @pl.when(c + 2 < NCHUNK)
            def _():
                dma(c + 2, k).start()
        return 0

    lax.fori_loop(0, NCHUNK // 2, chunk_pair, 0)

    pltpu.sync_copy(accm_v, pm.at[b, j])
    pltpu.sync_copy(acca_v, pa.at[b, j])


def _sc_partials(hid, msk_i32):
    mesh = plsc.VectorSubcoreMesh(
        core_axis_name="c", subcore_axis_name="s",
        num_cores=NC, num_subcores=NS)
    f = pl.kernel(
        _sc_body,
        out_type=(
            jax.ShapeDtypeStruct((B, WPB, D), jnp.float32),
            jax.ShapeDtypeStruct((B, WPB, D), jnp.float32),
            jax.ShapeDtypeStruct((B, WPB * LANES), jnp.float32),
        ),
        mesh=mesh,
        scratch_types=(
            pltpu.VMEM((RPW,), jnp.int32),
            pltpu.VMEM((RPW,), jnp.float32),
            pltpu.VMEM((LANES,), jnp.float32),
            pltpu.VMEM((D,), jnp.float32),
            pltpu.VMEM((D,), jnp.float32),
            pltpu.VMEM((CHUNK, D), jnp.float32),
            pltpu.VMEM((CHUNK, D), jnp.float32),
            pltpu.SemaphoreType.DMA,
            pltpu.SemaphoreType.DMA,
        ),
        name="utr_pool_sc_partials",
    )
    return f(hid, msk_i32)


def _finalize_body(pm_ref, pa_ref, pc_ref, out_ref):
    summ = jnp.sum(pm_ref[...], axis=1)
    suma = jnp.sum(pa_ref[...], axis=1)
    cnt = jnp.sum(pc_ref[...], axis=1)
    safe = jnp.maximum(cnt, 1.0)
    out_ref[...] = jnp.where((cnt > 0)[:, None],
                             summ / safe[:, None],
                             suma * (1.0 / L))


def _finalize(pm, pa, pc):
    return pl.pallas_call(
        _finalize_body,
        out_shape=jax.ShapeDtypeStruct((B, D), jnp.float32),
        in_specs=[
            pl.BlockSpec(memory_space=pltpu.VMEM),
            pl.BlockSpec(memory_space=pltpu.VMEM),
            pl.BlockSpec(memory_space=pltpu.VMEM),
        ],
        out_specs=pl.BlockSpec(memory_space=pltpu.VMEM),
        name="utr_pool_finalize",
    )(pm, pa, pc)


@jax.jit
def kernel(hidden_states, region_mask):
    msk_i32 = region_mask.astype(jnp.int32)
    pm, pa, pc = _sc_partials(hidden_states, msk_i32)
    return _finalize(pm, pa, pc)
